# baseline (speedup 1.0000x reference)
import jax
import jax.numpy as jnp
from jax import lax
from jax.experimental import pallas as pl
from jax.experimental.pallas import tpu as pltpu
from jax.experimental.pallas import tpu_sc as plsc

_N = 1000000
_H = 499712
_NUM_TYPES = 100
_TAB = 128
_LANES = 16
_NW = 32
_CH = 4096
_NB = 2
_C = 16384
_NCH = _C // _CH
_NGR = _NCH // _NB


def _make_body(start, length):
    def _body(z_hbm, tab_hbm, out_hbm,
              z0, z1, o0, o1, tab_v,
              sz0, sz1, so0, so1):
        zs, os = (z0, z1), (o0, o1)
        szs, sos = (sz0, sz1), (so0, so1)
        c = lax.axis_index("c")
        s = lax.axis_index("s")
        wid = s * 2 + c
        base = start + ((wid * (length - _C)) // (_NW - 1)) // 8 * 8
        obase = base - start
        pltpu.sync_copy(tab_hbm, tab_v)

        def start_in(k, b):
            off = base + k * _CH
            pltpu.async_copy(z_hbm.at[pl.ds(off, _CH)], zs[b], szs[b])

        for b in range(_NB):
            start_in(b, b)

        def group(g, carry):
            for b in range(_NB):
                k = g * _NB + b
                pltpu.make_async_copy(
                    z_hbm.at[pl.ds(0, _CH)], zs[b], szs[b]).wait()

                @pl.when(g > 0)
                def _():
                    pltpu.make_async_copy(
                        os[b], out_hbm.at[pl.ds(0, _CH)], sos[b]).wait()

                zb, ob = zs[b], os[b]

                @plsc.parallel_loop(0, _CH, step=_LANES, unroll=8)
                def _gather(i):
                    sl = pl.ds(i, _LANES)
                    ob[sl] = plsc.load_gather(tab_v, [zb[sl]])

                pltpu.async_copy(
                    ob, out_hbm.at[pl.ds(obase + k * _CH, _CH)], sos[b])

                @pl.when(g < _NGR - 1)
                def _():
                    start_in(k + _NB, b)
            return carry

        lax.fori_loop(0, _NGR, group, None)
        for b in range(_NB):
            pltpu.make_async_copy(
                os[b], out_hbm.at[pl.ds(0, _CH)], sos[b]).wait()

    return _body


def _make_run(start, length):
    mesh = plsc.VectorSubcoreMesh(core_axis_name="c", subcore_axis_name="s")
    return pl.kernel(
        _make_body(start, length),
        out_type=jax.ShapeDtypeStruct((length,), jnp.float32),
        mesh=mesh,
        compiler_params=pltpu.CompilerParams(needs_layout_passes=False),
        scratch_types=[
            pltpu.VMEM((_CH,), jnp.int32),
            pltpu.VMEM((_CH,), jnp.int32),
            pltpu.VMEM((_CH,), jnp.float32),
            pltpu.VMEM((_CH,), jnp.float32),
            pltpu.VMEM((_TAB,), jnp.float32),
            pltpu.SemaphoreType.DMA,
            pltpu.SemaphoreType.DMA,
            pltpu.SemaphoreType.DMA,
            pltpu.SemaphoreType.DMA,
        ],
    )


def kernel(x, z, pos, batch, atomref):
    del pos, batch
    tab = jnp.pad(atomref.reshape(_NUM_TYPES), (0, _TAB - _NUM_TYPES))
    g1 = _make_run(0, _H)(z, tab)
    g2 = _make_run(_H, _N - _H)(z, tab)
    o1 = x[:_H] + g1.reshape(_H, 1)
    o2 = x[_H:] + g2.reshape(_N - _H, 1)
    return jnp.concatenate([o1, o2], axis=0)


# speedup vs baseline: 1.6398x; 1.6398x over previous
import jax
import jax.numpy as jnp
from jax import lax
from jax.experimental import pallas as pl
from jax.experimental.pallas import tpu as pltpu
from jax.experimental.pallas import tpu_sc as plsc

_N = 1000000
_NUM_TYPES = 100
_LANES = 16
_NW = 32
_CH = 8192
_NB = 2
_C = 32768
_NCH = _C // _CH
_NGR = _NCH // _NB


def _body(z_hbm, tab_hbm, out_hbm,
          z0, z1, o0, o1, tab_v,
          sz0, sz1, so0, so1):
    zs, os = (z0, z1), (o0, o1)
    szs, sos = (sz0, sz1), (so0, so1)
    c = lax.axis_index("c")
    s = lax.axis_index("s")
    wid = s * 2 + c
    base = ((wid * (_N - _C)) // (_NW - 1)) // 8 * 8
    pltpu.sync_copy(tab_hbm, tab_v)

    def start_in(k, b):
        off = base + k * _CH
        pltpu.async_copy(z_hbm.at[pl.ds(off, _CH)], zs[b], szs[b])

    for b in range(_NB):
        start_in(b, b)

    def group(g, carry):
        for b in range(_NB):
            k = g * _NB + b
            pltpu.make_async_copy(
                z_hbm.at[pl.ds(0, _CH)], zs[b], szs[b]).wait()

            @pl.when(g > 0)
            def _():
                pltpu.make_async_copy(
                    os[b], out_hbm.at[pl.ds(0, _CH)], sos[b]).wait()

            zb, ob = zs[b], os[b]

            @plsc.parallel_loop(0, _CH, step=_LANES, unroll=16)
            def _gather(i):
                sl = pl.ds(i, _LANES)
                ob[sl] = plsc.load_gather(tab_v, [zb[sl]])

            pltpu.async_copy(
                ob, out_hbm.at[pl.ds(base + k * _CH, _CH)], sos[b])

            @pl.when(g < _NGR - 1)
            def _():
                start_in(k + _NB, b)
        return carry

    lax.fori_loop(0, _NGR, group, None)
    for b in range(_NB):
        pltpu.make_async_copy(
            os[b], out_hbm.at[pl.ds(0, _CH)], sos[b]).wait()


def kernel(x, z, pos, batch, atomref):
    del pos, batch
    mesh = plsc.VectorSubcoreMesh(core_axis_name="c", subcore_axis_name="s")
    run = pl.kernel(
        _body,
        out_type=jax.ShapeDtypeStruct((_N,), jnp.float32),
        mesh=mesh,
        compiler_params=pltpu.CompilerParams(needs_layout_passes=False),
        scratch_types=[
            pltpu.VMEM((_CH,), jnp.int32),
            pltpu.VMEM((_CH,), jnp.int32),
            pltpu.VMEM((_CH,), jnp.float32),
            pltpu.VMEM((_CH,), jnp.float32),
            pltpu.VMEM((_NUM_TYPES,), jnp.float32),
            pltpu.SemaphoreType.DMA,
            pltpu.SemaphoreType.DMA,
            pltpu.SemaphoreType.DMA,
            pltpu.SemaphoreType.DMA,
        ],
    )
    g = run(z, atomref.reshape(_NUM_TYPES))
    return x + g.reshape(_N, 1)


# speedup vs baseline: 1.6414x; 1.0010x over previous
import jax
import jax.numpy as jnp
from jax import lax
from jax.experimental import pallas as pl
from jax.experimental.pallas import tpu as pltpu
from jax.experimental.pallas import tpu_sc as plsc

_N = 1000000
_NUM_TYPES = 100
_LANES = 16
_NW = 32
_CH = 8192
_NB = 2
_C = 32768
_NCH = _C // _CH
_NGR = _NCH // _NB


def _body(z_hbm, tab_hbm, out_hbm,
          z0, z1, o0, o1, tab_v,
          sz0, sz1, so0, so1):
    zs, os = (z0, z1), (o0, o1)
    szs, sos = (sz0, sz1), (so0, so1)
    c = lax.axis_index("c")
    s = lax.axis_index("s")
    wid = s * 2 + c
    base = ((wid * (_N - _C)) // (_NW - 1)) // 8 * 8
    pltpu.sync_copy(tab_hbm, tab_v)

    def start_in(k, b):
        off = base + k * _CH
        pltpu.async_copy(z_hbm.at[pl.ds(off, _CH)], zs[b], szs[b])

    for b in range(_NB):
        start_in(b, b)

    def group(g, carry):
        for b in range(_NB):
            k = g * _NB + b
            pltpu.make_async_copy(
                z_hbm.at[pl.ds(0, _CH)], zs[b], szs[b]).wait()

            @pl.when(g > 0)
            def _():
                pltpu.make_async_copy(
                    os[b], out_hbm.at[pl.ds(0, _CH)], sos[b]).wait()

            zb, ob = zs[b], os[b]

            @plsc.parallel_loop(0, _CH, step=_LANES, unroll=16)
            def _gather(i):
                sl = pl.ds(i, _LANES)
                ob[sl] = plsc.load_gather(tab_v, [zb[sl]])

            pltpu.async_copy(
                ob, out_hbm.at[pl.ds(base + k * _CH, _CH)], sos[b])

            @pl.when(g < _NGR - 1)
            def _():
                start_in(k + _NB, b)
        return carry

    lax.fori_loop(0, _NGR, group, None)
    for b in range(_NB):
        pltpu.make_async_copy(
            os[b], out_hbm.at[pl.ds(0, _CH)], sos[b]).wait()


def kernel(x, z, pos, batch, atomref):
    del pos, batch
    mesh = plsc.VectorSubcoreMesh(core_axis_name="c", subcore_axis_name="s")
    run = pl.kernel(
        _body,
        out_type=jax.ShapeDtypeStruct((_N,), jnp.float32),
        mesh=mesh,
        compiler_params=pltpu.CompilerParams(needs_layout_passes=False),
        scratch_types=[
            pltpu.VMEM((_CH,), jnp.int32),
            pltpu.VMEM((_CH,), jnp.int32),
            pltpu.VMEM((_CH,), jnp.float32),
            pltpu.VMEM((_CH,), jnp.float32),
            pltpu.VMEM((_NUM_TYPES,), jnp.float32),
            pltpu.SemaphoreType.DMA,
            pltpu.SemaphoreType.DMA,
            pltpu.SemaphoreType.DMA,
            pltpu.SemaphoreType.DMA,
        ],
    )
    g = run(z, atomref.reshape(_NUM_TYPES))
    return x + jax.lax.broadcast_in_dim(g, (_N, 1), (0,))


# speedup vs baseline: 1.6479x; 1.0039x over previous
import jax
import jax.numpy as jnp
from jax import lax
from jax.experimental import pallas as pl
from jax.experimental.pallas import tpu as pltpu
from jax.experimental.pallas import tpu_sc as plsc

_N = 1000000
_NUM_TYPES = 100
_LANES = 16
_NW = 32
_CH = 4096
_NB = 4
_C = 32768
_NCH = _C // _CH
_NGR = _NCH // _NB


def _body(z_hbm, tab_hbm, out_hbm,
          z0, z1, z2, z3, o0, o1, o2, o3, tab_v,
          sz0, sz1, sz2, sz3, so0, so1, so2, so3):
    zs, os = (z0, z1, z2, z3), (o0, o1, o2, o3)
    szs, sos = (sz0, sz1, sz2, sz3), (so0, so1, so2, so3)
    c = lax.axis_index("c")
    s = lax.axis_index("s")
    wid = s * 2 + c
    base = ((wid * (_N - _C)) // (_NW - 1)) // 8 * 8
    pltpu.sync_copy(tab_hbm, tab_v)

    def start_in(k, b):
        off = base + k * _CH
        pltpu.async_copy(z_hbm.at[pl.ds(off, _CH)], zs[b], szs[b])

    for b in range(_NB):
        start_in(b, b)

    def group(g, carry):
        for b in range(_NB):
            k = g * _NB + b
            pltpu.make_async_copy(
                z_hbm.at[pl.ds(0, _CH)], zs[b], szs[b]).wait()

            @pl.when(g > 0)
            def _():
                pltpu.make_async_copy(
                    os[b], out_hbm.at[pl.ds(0, _CH)], sos[b]).wait()

            zb, ob = zs[b], os[b]

            @plsc.parallel_loop(0, _CH, step=_LANES, unroll=16)
            def _gather(i):
                sl = pl.ds(i, _LANES)
                ob[sl] = plsc.load_gather(tab_v, [zb[sl]])

            pltpu.async_copy(
                ob, out_hbm.at[pl.ds(base + k * _CH, _CH)], sos[b])

            @pl.when(g < _NGR - 1)
            def _():
                start_in(k + _NB, b)
        return carry

    lax.fori_loop(0, _NGR, group, None)
    for b in range(_NB):
        pltpu.make_async_copy(
            os[b], out_hbm.at[pl.ds(0, _CH)], sos[b]).wait()


def kernel(x, z, pos, batch, atomref):
    del pos, batch
    mesh = plsc.VectorSubcoreMesh(core_axis_name="c", subcore_axis_name="s")
    run = pl.kernel(
        _body,
        out_type=jax.ShapeDtypeStruct((_N,), jnp.float32),
        mesh=mesh,
        compiler_params=pltpu.CompilerParams(needs_layout_passes=False),
        scratch_types=(
            [pltpu.VMEM((_CH,), jnp.int32)] * 4
            + [pltpu.VMEM((_CH,), jnp.float32)] * 4
            + [pltpu.VMEM((_NUM_TYPES,), jnp.float32)]
            + [pltpu.SemaphoreType.DMA] * 8
        ),
    )
    g = run(z, atomref.reshape(_NUM_TYPES))
    return x + jax.lax.broadcast_in_dim(g, (_N, 1), (0,))
